# Initial kernel scaffold; baseline (speedup 1.0000x reference)
import jax
import jax.numpy as jnp
from jax.experimental import pallas as pl


def kernel(x, edge_index, W1, as1, ad1, b1, W2, as2, ad2, b2, W3, as3, ad3, b3, hW, hb):
    raise NotImplementedError("write your pallas kernel here")


# speedup vs baseline: 40.7457x; 40.7457x over previous
import functools

import jax
import jax.numpy as jnp
from jax import lax
from jax.experimental import pallas as pl
from jax.experimental.pallas import tpu as pltpu
from jax.experimental.pallas import tpu_sc as plsc

NC, NS, LANES = 2, 16, 16
NW = NC * NS
CH = 1024
SUB = 128
TW = 16
BN = 512


def _tc_pre_body(x_ref, w_ref, am_ref, ad_ref, h_ref, as_ref, adst_ref):
    xb = x_ref[...]
    hb = jnp.dot(xb, w_ref[...], preferred_element_type=jnp.float32)
    h_ref[...] = hb
    as_ref[...] = jnp.dot(hb, am_ref[...], preferred_element_type=jnp.float32)
    adst_ref[...] = jnp.dot(hb, ad_ref[...], preferred_element_type=jnp.float32)


def _tc_pre(xp, W, am, ad, n1pad):
    inc = xp.shape[1]
    f = W.shape[1]
    grid = (n1pad // BN,)
    return pl.pallas_call(
        _tc_pre_body,
        grid=grid,
        in_specs=[
            pl.BlockSpec((BN, inc), lambda i: (i, 0)),
            pl.BlockSpec((inc, f), lambda i: (0, 0)),
            pl.BlockSpec((f, TW), lambda i: (0, 0)),
            pl.BlockSpec((f, TW), lambda i: (0, 0)),
        ],
        out_specs=[
            pl.BlockSpec((BN, f), lambda i: (i, 0)),
            pl.BlockSpec((BN, TW), lambda i: (i, 0)),
            pl.BlockSpec((BN, TW), lambda i: (i, 0)),
        ],
        out_shape=[
            jax.ShapeDtypeStruct((n1pad, f), jnp.float32),
            jax.ShapeDtypeStruct((n1pad, TW), jnp.float32),
            jax.ShapeDtypeStruct((n1pad, TW), jnp.float32),
        ],
    )(xp, W, am, ad)


def _make_mid_body(H):
    def body(*refs):
        o_refs = refs[:H]
        s2_ref, b_ref, w_ref, am_ref, ad_ref, h_ref, as_ref, adst_ref = refs[H:]
        s2 = s2_ref[...]
        s = s2[0] + s2[1]
        parts = []
        for hd in range(H):
            o = o_refs[hd][...]
            acc = o[0] + o[1]
            parts.append(acc / (s[:, hd:hd + 1] + 1e-16))
        z = (jnp.concatenate(parts, axis=1) if H > 1 else parts[0]) + b_ref[...]
        xb = jnp.where(z > 0, z, jnp.exp(z) - 1.0)
        hb = jnp.dot(xb, w_ref[...], preferred_element_type=jnp.float32)
        h_ref[...] = hb
        as_ref[...] = jnp.dot(hb, am_ref[...], preferred_element_type=jnp.float32)
        adst_ref[...] = jnp.dot(hb, ad_ref[...], preferred_element_type=jnp.float32)
    return body


def _tc_mid(o_list, s2, b2d, W, am, ad, n1pad):
    H = len(o_list)
    f = W.shape[1]
    fin = W.shape[0]
    grid = (n1pad // BN,)
    in_specs = [pl.BlockSpec((NC, BN, 32), lambda i: (0, i, 0)) for _ in range(H)]
    in_specs += [
        pl.BlockSpec((NC, BN, TW), lambda i: (0, i, 0)),
        pl.BlockSpec((1, fin), lambda i: (0, 0)),
        pl.BlockSpec((fin, f), lambda i: (0, 0)),
        pl.BlockSpec((f, TW), lambda i: (0, 0)),
        pl.BlockSpec((f, TW), lambda i: (0, 0)),
    ]
    return pl.pallas_call(
        _make_mid_body(H),
        grid=grid,
        in_specs=in_specs,
        out_specs=[
            pl.BlockSpec((BN, f), lambda i: (i, 0)),
            pl.BlockSpec((BN, TW), lambda i: (i, 0)),
            pl.BlockSpec((BN, TW), lambda i: (i, 0)),
        ],
        out_shape=[
            jax.ShapeDtypeStruct((n1pad, f), jnp.float32),
            jax.ShapeDtypeStruct((n1pad, TW), jnp.float32),
            jax.ShapeDtypeStruct((n1pad, TW), jnp.float32),
        ],
    )(*o_list, s2, b2d, W, am, ad)


def _fin3_body(o_ref, s2_ref, b_ref, hw_ref, hb_ref, y_ref):
    o = o_ref[...]
    acc = o[0] + o[1]
    s2 = s2_ref[...]
    s = (s2[0] + s2[1])[:, 0:1] + 1e-16
    z = acc / s + b_ref[...]
    h3 = jnp.where(z > 0, z, jnp.exp(z) - 1.0)
    y_ref[...] = jnp.dot(h3, hw_ref[...], preferred_element_type=jnp.float32) + hb_ref[...]


def _tc_fin3(o3, s3, b2d, hW, hb2d, n1pad):
    grid = (n1pad // BN,)
    return pl.pallas_call(
        _fin3_body,
        grid=grid,
        in_specs=[
            pl.BlockSpec((NC, BN, 32), lambda i: (0, i, 0)),
            pl.BlockSpec((NC, BN, TW), lambda i: (0, i, 0)),
            pl.BlockSpec((1, 32), lambda i: (0, 0)),
            pl.BlockSpec((32, 1), lambda i: (0, 0)),
            pl.BlockSpec((1, 1), lambda i: (0, 0)),
        ],
        out_specs=pl.BlockSpec((BN, 1), lambda i: (i, 0)),
        out_shape=jax.ShapeDtypeStruct((n1pad, 1), jnp.float32),
    )(o3, s3, b2d, hW, hb2d)


def _make_pass_a(H, n1pad, ew, nchunk, e_pad):
    rps = n1pad // NS
    mesh = plsc.VectorSubcoreMesh(core_axis_name="c", subcore_axis_name="s")

    @functools.partial(
        pl.kernel,
        out_type=[
            jax.ShapeDtypeStruct((e_pad, TW), jnp.float32),
            jax.ShapeDtypeStruct((NC, n1pad, TW), jnp.float32),
        ],
        mesh=mesh,
        compiler_params=pltpu.CompilerParams(use_tc_tiling_on_sc=False),
        scratch_types=[
            pltpu.VMEM_SHARED((n1pad, TW), jnp.float32),
            pltpu.VMEM((CH // SUB, SUB), jnp.int32),
            pltpu.VMEM((CH // SUB, SUB), jnp.int32),
            pltpu.VMEM((SUB, TW), jnp.float32),
            pltpu.VMEM((SUB, TW), jnp.float32),
            pltpu.VMEM((SUB, TW), jnp.float32),
            pltpu.SemaphoreType.DMA,
        ],
    )
    def pass_a(src2d, dst2d, a_s, a_d, z16, p_t, s_out,
               s_sh, srcv, dstv, asrows, adrows, pscb, sem):
        cid = lax.axis_index("c")
        sid = lax.axis_index("s")
        wid = sid * NC + cid
        pltpu.sync_copy(z16, s_sh.at[pl.ds(pl.multiple_of(sid * rps, 8), rps)])
        plsc.subcore_barrier()

        def chunk(ci, carry):
            base = wid * ew + ci * CH
            rb = pl.multiple_of(base // SUB, CH // SUB)
            pltpu.sync_copy(src2d.at[pl.ds(rb, CH // SUB)], srcv)
            pltpu.sync_copy(dst2d.at[pl.ds(rb, CH // SUB)], dstv)

            def sub(j, scarry):
                c1 = pltpu.async_copy(a_s.at[srcv.at[j]], asrows, sem)
                c2 = pltpu.async_copy(a_d.at[dstv.at[j]], adrows, sem)
                c1.wait()
                c2.wait()
                for r in range(SUB):
                    e = asrows[r, :] + adrows[r, :]
                    e = jnp.maximum(e, 0.2 * e)
                    pscb[r, :] = jnp.exp(e)
                pltpu.sync_copy(pscb, s_sh.at[dstv.at[j]], add=True)
                off = pl.multiple_of(base + j * SUB, SUB)
                pltpu.sync_copy(pscb, p_t.at[pl.ds(off, SUB)])
                return scarry

            lax.fori_loop(0, CH // SUB, sub, 0)
            return carry

        lax.fori_loop(0, nchunk, chunk, 0)
        plsc.subcore_barrier()
        pltpu.sync_copy(s_sh.at[pl.ds(pl.multiple_of(sid * rps, 8), rps)],
                        s_out.at[cid, pl.ds(pl.multiple_of(sid * rps, 8), rps)])

    return pass_a


def _make_pass_b(H, hd, n1pad, ew, nchunk):
    rps = n1pad // NS
    mesh = plsc.VectorSubcoreMesh(core_axis_name="c", subcore_axis_name="s")

    @functools.partial(
        pl.kernel,
        out_type=jax.ShapeDtypeStruct((NC, n1pad, 32), jnp.float32),
        mesh=mesh,
        compiler_params=pltpu.CompilerParams(use_tc_tiling_on_sc=False),
        scratch_types=[
            pltpu.VMEM_SHARED((n1pad, 32), jnp.float32),
            pltpu.VMEM((CH // SUB, SUB), jnp.int32),
            pltpu.VMEM((CH // SUB, SUB), jnp.int32),
            pltpu.VMEM((CH // SUB, SUB), jnp.int32),
            pltpu.VMEM((SUB, 32), jnp.float32),
            pltpu.VMEM((SUB, 32), jnp.float32),
            pltpu.VMEM((CH, TW), jnp.float32),
            pltpu.SemaphoreType.DMA,
        ],
    )
    def pass_b(src2d, dst2d, p_t, htab, z32, o_out,
               acc_sh, srcv, dstv, idxg, hrows, msg, pvr, sem):
        cid = lax.axis_index("c")
        sid = lax.axis_index("s")
        wid = sid * NC + cid
        pltpu.sync_copy(z32, acc_sh.at[pl.ds(pl.multiple_of(sid * rps, 8), rps)])
        plsc.subcore_barrier()

        def chunk(ci, carry):
            base = wid * ew + ci * CH
            rb = pl.multiple_of(base // SUB, CH // SUB)
            pltpu.sync_copy(src2d.at[pl.ds(rb, CH // SUB)], srcv)
            pltpu.sync_copy(dst2d.at[pl.ds(rb, CH // SUB)], dstv)
            pltpu.sync_copy(p_t.at[pl.ds(pl.multiple_of(base, CH), CH)], pvr)

            def sub(j, scarry):
                if H > 1:
                    for k in range(SUB // LANES):
                        sl = pl.ds(k * LANES, LANES)
                        idxg[j, sl] = srcv[j, sl] * H + hd
                    gidx = idxg.at[j]
                else:
                    gidx = srcv.at[j]
                pltpu.async_copy(htab.at[gidx], hrows, sem).wait()
                rbase = j * SUB
                for g in range(SUB // LANES):
                    for e in range(LANES):
                        r = g * LANES + e
                        prow = pvr[rbase + r, :]
                        pb = jnp.broadcast_to(prow[hd], (LANES,))
                        msg[r, pl.ds(0, 16)] = hrows[r, pl.ds(0, 16)] * pb
                        msg[r, pl.ds(16, 16)] = hrows[r, pl.ds(16, 16)] * pb
                pltpu.sync_copy(msg, acc_sh.at[dstv.at[j]], add=True)
                return scarry

            lax.fori_loop(0, CH // SUB, sub, 0)
            return carry

        lax.fori_loop(0, nchunk, chunk, 0)
        plsc.subcore_barrier()
        pltpu.sync_copy(acc_sh.at[pl.ds(pl.multiple_of(sid * rps, 8), rps)],
                        o_out.at[cid, pl.ds(pl.multiple_of(sid * rps, 8), rps)])

    return pass_b


def _amat(a):
    H, C = a.shape[1], a.shape[2]
    m = a[0][:, :, None] * jnp.eye(H, dtype=jnp.float32)[:, None, :]
    m = m.reshape(H * C, H)
    return jnp.pad(m, ((0, 0), (0, TW - H)))


def kernel(x, edge_index, W1, as1, ad1, b1, W2, as2, ad2, b2,
           W3, as3, ad3, b3, hW, hb):
    N = x.shape[0]
    E = edge_index.shape[1]
    n1pad = ((N + 1 + BN - 1) // BN) * BN
    if n1pad % (NS * 16):
        n1pad = ((n1pad + NS * 16 - 1) // (NS * 16)) * (NS * 16)
    rps = n1pad // NS
    etot = E + N
    ew = ((etot + NW * CH - 1) // (NW * CH)) * CH
    nchunk = ew // CH
    e_pad = ew * NW

    loop = jnp.arange(N, dtype=jnp.int32)
    src = jnp.concatenate([edge_index[0], loop])
    dst = jnp.concatenate([edge_index[1], loop])
    fill = jnp.full((e_pad - etot,), N, jnp.int32)
    src2d = jnp.concatenate([src, fill]).reshape(e_pad // SUB, SUB)
    dst2d = jnp.concatenate([dst, fill]).reshape(e_pad // SUB, SUB)
    xp = jnp.concatenate(
        [x, jnp.zeros((n1pad - N, x.shape[1]), jnp.float32)])
    z32 = jnp.zeros((rps, 32), jnp.float32)
    z16 = jnp.zeros((rps, TW), jnp.float32)

    pass_a4 = _make_pass_a(4, n1pad, ew, nchunk, e_pad)
    pass_a1 = _make_pass_a(1, n1pad, ew, nchunk, e_pad)
    pass_b4 = [_make_pass_b(4, hd, n1pad, ew, nchunk) for hd in range(4)]
    pass_b1 = _make_pass_b(1, 0, n1pad, ew, nchunk)

    h1, s1_, d1_ = _tc_pre(xp, W1, _amat(as1), _amat(ad1), n1pad)
    pt1, s1 = pass_a4(src2d, dst2d, s1_, d1_, z16)
    htab1 = h1.reshape(n1pad * 4, 32)
    o1 = [pass_b4[hd](src2d, dst2d, pt1, htab1, z32) for hd in range(4)]

    h2, s2_, d2_ = _tc_mid(o1, s1, b1.reshape(1, -1), W2,
                           _amat(as2), _amat(ad2), n1pad)
    pt2, s2 = pass_a4(src2d, dst2d, s2_, d2_, z16)
    htab2 = h2.reshape(n1pad * 4, 32)
    o2 = [pass_b4[hd](src2d, dst2d, pt2, htab2, z32) for hd in range(4)]

    h3, s3_, d3_ = _tc_mid(o2, s2, b2.reshape(1, -1), W3,
                           _amat(as3), _amat(ad3), n1pad)
    pt3, s3 = pass_a1(src2d, dst2d, s3_, d3_, z16)
    o3 = pass_b1(src2d, dst2d, pt3, h3, z32)

    y = _tc_fin3(o3, s3, b3.reshape(1, -1), hW, hb.reshape(1, -1), n1pad)
    return y[:N, 0]
